# baseline (speedup 1.0000x reference)
import functools

import jax
import jax.numpy as jnp
from jax.experimental import pallas as pl
from jax.experimental.pallas import tpu as pltpu
from jax.experimental.pallas import tpu_sc as plsc

_DIM = 768
_HALF = _DIM // 2
_HIDDEN = 2048
_BLK = 512
_CHUNK = 8192
_NBLK = _CHUNK // _BLK + 1
_PAD_CHUNK = _NBLK * _BLK
_W = 128


@functools.cache
def _vector_mesh():
    return plsc.VectorSubcoreMesh(
        core_axis_name="core", subcore_axis_name="subcore")


def _routing_kernel(ids_ref, dest2_ref, eblk_ref):
    ind1 = ids_ref[0]
    ind0 = 1.0 - ind1
    r128 = jax.lax.broadcasted_iota(jnp.int32, (128, 128), 0)
    c128 = jax.lax.broadcasted_iota(jnp.int32, (128, 128), 1)
    upper = (r128 <= c128).astype(jnp.bfloat16)
    r64 = jax.lax.broadcasted_iota(jnp.int32, (64, 64), 0)
    c64 = jax.lax.broadcasted_iota(jnp.int32, (64, 64), 1)
    strict_lower = (r64 > c64).astype(jnp.bfloat16)

    def excl_prefix(ind):
        cs = jax.lax.dot_general(ind.astype(jnp.bfloat16), upper,
                                 (((1,), (0,)), ((), ())),
                                 preferred_element_type=jnp.float32)
        tot = cs[:, 127:128]
        rp = jax.lax.dot_general(strict_lower, tot.astype(jnp.bfloat16),
                                 (((1,), (0,)), ((), ())),
                                 preferred_element_type=jnp.float32)
        return rp + cs - ind, tot, rp

    g0, tot0, rp0 = excl_prefix(ind0)
    g1, _, _ = excl_prefix(ind1)
    n0 = (rp0[63:64, 0:1] + tot0[63:64, 0:1]).astype(jnp.int32)
    nb0 = (n0 + (_BLK - 1)) >> 9
    seg1_start = nb0 << 9
    dest = jnp.where(ind1 > 0.5,
                     seg1_start + g1.astype(jnp.int32),
                     g0.astype(jnp.int32))
    dest2_ref[0, 0] = dest
    dest2_ref[0, 1] = dest + _PAD_CHUNK
    lane = jax.lax.broadcasted_iota(jnp.int32, (1, 128), 1)
    eblk_ref[0] = (lane >= nb0).astype(jnp.int32)


def _route(ids_f32):
    n_chunks = ids_f32.shape[0]
    return pl.pallas_call(
        _routing_kernel,
        grid=(n_chunks,),
        in_specs=[pl.BlockSpec((1, 64, 128), lambda c: (c, 0, 0))],
        out_specs=[pl.BlockSpec((1, 2, 64, 128), lambda c: (c, 0, 0, 0)),
                   pl.BlockSpec((1, 1, 128), lambda c: (c, 0, 0))],
        out_shape=[jax.ShapeDtypeStruct((n_chunks, 2, 64, 128), jnp.int32),
                   jax.ShapeDtypeStruct((n_chunks, 1, 128), jnp.int32)],
    )(ids_f32)


def _dispatch(xf, dest2_c, chunk_idx):
    blk_off = chunk_idx * (_CHUNK // _W)

    @pl.kernel(out_type=jax.ShapeDtypeStruct((2 * _PAD_CHUNK, _HALF),
                                             jnp.float32),
               mesh=_vector_mesh())
    def dispatch_kernel(x_hbm, d_hbm, o_hbm):
        def body(x_vmem, i_vmem):
            pltpu.sync_copy(x_vmem, o_hbm.at[i_vmem.at[0]])

        pltpu.emit_pipeline(
            body,
            grid=(2, _CHUNK // _W),
            in_specs=[pl.BlockSpec((_W, _HALF),
                                   lambda p, i: (blk_off + i, p)),
                      pl.BlockSpec((1, _W), lambda p, i: (p, i))],
            out_specs=[],
            core_axis_name=("core", "subcore"),
            dimension_semantics=(pltpu.PARALLEL, pltpu.PARALLEL),
        )(x_hbm, d_hbm)

    return dispatch_kernel(xf, dest2_c)


def _return_gather(out_sorted_c, dest2_c):
    @pl.kernel(out_type=jax.ShapeDtypeStruct((_CHUNK, _DIM), jnp.float32),
               mesh=_vector_mesh())
    def return_kernel(src_hbm, d_hbm, o_hbm):
        def body(i_vmem, o_vmem):
            pltpu.sync_copy(src_hbm.at[i_vmem.at[0]], o_vmem)

        pltpu.emit_pipeline(
            body,
            grid=(2, _CHUNK // _W),
            in_specs=[pl.BlockSpec((1, _W), lambda p, i: (p, i))],
            out_specs=[pl.BlockSpec((_W, _HALF), lambda p, i: (i, p))],
            core_axis_name=("core", "subcore"),
            dimension_semantics=(pltpu.PARALLEL, pltpu.PARALLEL),
        )(d_hbm, o_hbm)

    return return_kernel(out_sorted_c, dest2_c)


def _ffn_kernel(eblk_ref, xlo_ref, xhi_ref, w1a_ref, w1b_ref, w3a_ref,
                w3b_ref, w2a_ref, w2b_ref, o_ref, h_ref):
    jc = pl.program_id(1)

    @pl.when(jc == 0)
    def _compute_h():
        xlo = xlo_ref[...].astype(jnp.bfloat16)
        xhi = xhi_ref[...].astype(jnp.bfloat16)
        dims = (((1,), (1,)), ((), ()))
        x1 = (jax.lax.dot_general(xlo, w1a_ref[0], dims,
                                  preferred_element_type=jnp.float32)
              + jax.lax.dot_general(xhi, w1b_ref[0], dims,
                                    preferred_element_type=jnp.float32))
        x3 = (jax.lax.dot_general(xlo, w3a_ref[0], dims,
                                  preferred_element_type=jnp.float32)
              + jax.lax.dot_general(xhi, w3b_ref[0], dims,
                                    preferred_element_type=jnp.float32))
        h_ref[...] = (x1 * jax.lax.logistic(x1) * x3).astype(jnp.bfloat16)
        o_ref[...] = jax.lax.dot_general(
            h_ref[...], w2a_ref[0], (((1,), (1,)), ((), ())),
            preferred_element_type=jnp.float32)

    @pl.when(jc == 1)
    def _second_half():
        o_ref[...] = jax.lax.dot_general(
            h_ref[...], w2b_ref[0], (((1,), (1,)), ((), ())),
            preferred_element_type=jnp.float32)


def _ffn(x_sorted_c, eblk_c, w1a, w1b, w3a, w3b, w2a, w2b):
    grid_spec = pltpu.PrefetchScalarGridSpec(
        num_scalar_prefetch=1,
        grid=(_NBLK, 2),
        in_specs=[
            pl.BlockSpec((_BLK, _HALF), lambda j, jc, s: (j, 0)),
            pl.BlockSpec((_BLK, _HALF), lambda j, jc, s: (_NBLK + j, 0)),
            pl.BlockSpec((1, _HIDDEN, _HALF), lambda j, jc, s: (s[j], 0, 0)),
            pl.BlockSpec((1, _HIDDEN, _HALF), lambda j, jc, s: (s[j], 0, 0)),
            pl.BlockSpec((1, _HIDDEN, _HALF), lambda j, jc, s: (s[j], 0, 0)),
            pl.BlockSpec((1, _HIDDEN, _HALF), lambda j, jc, s: (s[j], 0, 0)),
            pl.BlockSpec((1, _HALF, _HIDDEN), lambda j, jc, s: (s[j], 0, 0)),
            pl.BlockSpec((1, _HALF, _HIDDEN), lambda j, jc, s: (s[j], 0, 0)),
        ],
        out_specs=pl.BlockSpec((_BLK, _HALF),
                               lambda j, jc, s: (jc * _NBLK + j, 0)),
        scratch_shapes=[pltpu.VMEM((_BLK, _HIDDEN), jnp.bfloat16)],
    )
    return pl.pallas_call(
        _ffn_kernel,
        grid_spec=grid_spec,
        out_shape=jax.ShapeDtypeStruct((2 * _PAD_CHUNK, _HALF), jnp.float32),
    )(eblk_c, x_sorted_c, x_sorted_c, w1a, w1b, w3a, w3b, w2a, w2b)


def kernel(x, modality_ids, W1, W2, W3):
    bsz, seq_len, dim = x.shape
    n_tok = bsz * seq_len
    n_chunks = n_tok // _CHUNK
    xf = x.reshape(n_tok, dim)
    ids_f32 = modality_ids.astype(jnp.float32).reshape(n_chunks, 64, 128)
    w1 = W1.astype(jnp.bfloat16)
    w3 = W3.astype(jnp.bfloat16)
    w2 = W2.astype(jnp.bfloat16)
    w1a, w1b = w1[:, :, :_HALF], w1[:, :, _HALF:]
    w3a, w3b = w3[:, :, :_HALF], w3[:, :, _HALF:]
    w2a, w2b = w2[:, :_HALF, :], w2[:, _HALF:, :]

    dest2, eblk = _route(ids_f32)
    dest2_flat = dest2.reshape(n_chunks, 2, _CHUNK)
    eblk_flat = eblk.reshape(n_chunks, 128)

    outs = []
    for c in range(n_chunks):
        x_sorted_c = _dispatch(xf, dest2_flat[c], c)
        out_sorted_c = _ffn(x_sorted_c, eblk_flat[c],
                            w1a, w1b, w3a, w3b, w2a, w2b)
        outs.append(_return_gather(out_sorted_c, dest2_flat[c]))
    out = jnp.stack(outs)
    return out.reshape(bsz, seq_len, dim)


# speedup vs baseline: 1.0779x; 1.0779x over previous
import functools

import jax
import jax.numpy as jnp
from jax.experimental import pallas as pl
from jax.experimental.pallas import tpu as pltpu
from jax.experimental.pallas import tpu_sc as plsc

_DIM = 768
_HALF = _DIM // 2
_HIDDEN = 2048
_BLK = 512
_CHUNK = 8192
_NBLK = _CHUNK // _BLK + 1
_PAD_CHUNK = _NBLK * _BLK
_W = 128


@functools.cache
def _vector_mesh():
    return plsc.VectorSubcoreMesh(
        core_axis_name="core", subcore_axis_name="subcore")


def _routing_kernel(ids_ref, dest2_ref, eblk_ref):
    ind1 = ids_ref[0]
    ind0 = 1.0 - ind1
    r128 = jax.lax.broadcasted_iota(jnp.int32, (128, 128), 0)
    c128 = jax.lax.broadcasted_iota(jnp.int32, (128, 128), 1)
    upper = (r128 <= c128).astype(jnp.bfloat16)
    r64 = jax.lax.broadcasted_iota(jnp.int32, (64, 64), 0)
    c64 = jax.lax.broadcasted_iota(jnp.int32, (64, 64), 1)
    strict_lower = (r64 > c64).astype(jnp.bfloat16)

    def excl_prefix(ind):
        cs = jax.lax.dot_general(ind.astype(jnp.bfloat16), upper,
                                 (((1,), (0,)), ((), ())),
                                 preferred_element_type=jnp.float32)
        tot = cs[:, 127:128]
        rp = jax.lax.dot_general(strict_lower, tot.astype(jnp.bfloat16),
                                 (((1,), (0,)), ((), ())),
                                 preferred_element_type=jnp.float32)
        return rp + cs - ind, tot, rp

    g0, tot0, rp0 = excl_prefix(ind0)
    g1, _, _ = excl_prefix(ind1)
    n0 = (rp0[63:64, 0:1] + tot0[63:64, 0:1]).astype(jnp.int32)
    nb0 = (n0 + (_BLK - 1)) >> 9
    seg1_start = nb0 << 9
    dest = jnp.where(ind1 > 0.5,
                     seg1_start + g1.astype(jnp.int32),
                     g0.astype(jnp.int32))
    dest2_ref[0, 0] = dest
    dest2_ref[0, 1] = dest + _PAD_CHUNK
    lane = jax.lax.broadcasted_iota(jnp.int32, (1, 128), 1)
    eblk_ref[0] = (lane >= nb0).astype(jnp.int32)


def _route(ids_f32):
    n_chunks = ids_f32.shape[0]
    return pl.pallas_call(
        _routing_kernel,
        grid=(n_chunks,),
        in_specs=[pl.BlockSpec((1, 64, 128), lambda c: (c, 0, 0))],
        out_specs=[pl.BlockSpec((1, 2, 64, 128), lambda c: (c, 0, 0, 0)),
                   pl.BlockSpec((1, 1, 128), lambda c: (c, 0, 0))],
        out_shape=[jax.ShapeDtypeStruct((n_chunks, 2, 64, 128), jnp.int32),
                   jax.ShapeDtypeStruct((n_chunks, 1, 128), jnp.int32)],
    )(ids_f32)


def _dispatch(xf, dest2_c, chunk_idx):
    blk_off = chunk_idx * (_CHUNK // _W)

    @pl.kernel(out_type=jax.ShapeDtypeStruct((2 * _PAD_CHUNK, _HALF),
                                             jnp.float32),
               mesh=_vector_mesh())
    def dispatch_kernel(x_hbm, d_hbm, o_hbm):
        def body(x_vmem, i_vmem):
            pltpu.sync_copy(x_vmem, o_hbm.at[i_vmem.at[0]])

        pltpu.emit_pipeline(
            body,
            grid=(2, _CHUNK // _W),
            in_specs=[pl.BlockSpec((_W, _HALF),
                                   lambda p, i: (blk_off + i, p)),
                      pl.BlockSpec((1, _W), lambda p, i: (p, i))],
            out_specs=[],
            core_axis_name=("core", "subcore"),
            dimension_semantics=(pltpu.PARALLEL, pltpu.PARALLEL),
        )(x_hbm, d_hbm)

    return dispatch_kernel(xf, dest2_c)


def _return_gather(out_plane_c, dest_c):
    @pl.kernel(out_type=jax.ShapeDtypeStruct((_CHUNK, _HALF), jnp.float32),
               mesh=_vector_mesh())
    def return_kernel(src_hbm, d_hbm, o_hbm):
        def body(i_vmem, o_vmem):
            pltpu.sync_copy(src_hbm.at[i_vmem.at[0]], o_vmem)

        pltpu.emit_pipeline(
            body,
            grid=(_CHUNK // _W,),
            in_specs=[pl.BlockSpec((1, _W), lambda i: (0, i))],
            out_specs=[pl.BlockSpec((_W, _HALF), lambda i: (i, 0))],
            core_axis_name=("core", "subcore"),
            dimension_semantics=(pltpu.PARALLEL,),
        )(d_hbm, o_hbm)

    return return_kernel(out_plane_c, dest_c)


def _ffn_kernel(eblk_ref, xlo_ref, xhi_ref, w1a_ref, w1b_ref, w3a_ref,
                w3b_ref, w2a_ref, w2b_ref, olo_ref, ohi_ref):
    xlo = xlo_ref[...].astype(jnp.bfloat16)
    xhi = xhi_ref[...].astype(jnp.bfloat16)
    dims = (((1,), (1,)), ((), ()))
    x1 = (jax.lax.dot_general(xlo, w1a_ref[0], dims,
                              preferred_element_type=jnp.float32)
          + jax.lax.dot_general(xhi, w1b_ref[0], dims,
                                preferred_element_type=jnp.float32))
    x3 = (jax.lax.dot_general(xlo, w3a_ref[0], dims,
                              preferred_element_type=jnp.float32)
          + jax.lax.dot_general(xhi, w3b_ref[0], dims,
                                preferred_element_type=jnp.float32))
    h = (x1 * jax.lax.logistic(x1) * x3).astype(jnp.bfloat16)
    olo_ref[...] = jax.lax.dot_general(h, w2a_ref[0], dims,
                                       preferred_element_type=jnp.float32)
    ohi_ref[...] = jax.lax.dot_general(h, w2b_ref[0], dims,
                                       preferred_element_type=jnp.float32)


def _ffn(x_sorted_c, eblk_c, w1a, w1b, w3a, w3b, w2a, w2b):
    grid_spec = pltpu.PrefetchScalarGridSpec(
        num_scalar_prefetch=1,
        grid=(_NBLK,),
        in_specs=[
            pl.BlockSpec((_BLK, _HALF), lambda j, s: (j, 0)),
            pl.BlockSpec((_BLK, _HALF), lambda j, s: (_NBLK + j, 0)),
            pl.BlockSpec((1, _HIDDEN, _HALF), lambda j, s: (s[j], 0, 0)),
            pl.BlockSpec((1, _HIDDEN, _HALF), lambda j, s: (s[j], 0, 0)),
            pl.BlockSpec((1, _HIDDEN, _HALF), lambda j, s: (s[j], 0, 0)),
            pl.BlockSpec((1, _HIDDEN, _HALF), lambda j, s: (s[j], 0, 0)),
            pl.BlockSpec((1, _HALF, _HIDDEN), lambda j, s: (s[j], 0, 0)),
            pl.BlockSpec((1, _HALF, _HIDDEN), lambda j, s: (s[j], 0, 0)),
        ],
        out_specs=[pl.BlockSpec((_BLK, _HALF), lambda j, s: (j, 0)),
                   pl.BlockSpec((_BLK, _HALF), lambda j, s: (j, 0))],
    )
    return pl.pallas_call(
        _ffn_kernel,
        grid_spec=grid_spec,
        out_shape=[jax.ShapeDtypeStruct((_PAD_CHUNK, _HALF), jnp.float32),
                   jax.ShapeDtypeStruct((_PAD_CHUNK, _HALF), jnp.float32)],
    )(eblk_c, x_sorted_c, x_sorted_c, w1a, w1b, w3a, w3b, w2a, w2b)


def kernel(x, modality_ids, W1, W2, W3):
    bsz, seq_len, dim = x.shape
    n_tok = bsz * seq_len
    n_chunks = n_tok // _CHUNK
    xf = x.reshape(n_tok, dim)
    ids_f32 = modality_ids.astype(jnp.float32).reshape(n_chunks, 64, 128)
    w1 = W1.astype(jnp.bfloat16)
    w3 = W3.astype(jnp.bfloat16)
    w2 = W2.astype(jnp.bfloat16)
    w1a, w1b = w1[:, :, :_HALF], w1[:, :, _HALF:]
    w3a, w3b = w3[:, :, :_HALF], w3[:, :, _HALF:]
    w2a, w2b = w2[:, :_HALF, :], w2[:, _HALF:, :]

    dest2, eblk = _route(ids_f32)
    dest2_flat = dest2.reshape(n_chunks, 2, _CHUNK)
    eblk_flat = eblk.reshape(n_chunks, 128)

    lo_outs, hi_outs = [], []
    for c in range(n_chunks):
        x_sorted_c = _dispatch(xf, dest2_flat[c], c)
        out_lo_c, out_hi_c = _ffn(x_sorted_c, eblk_flat[c],
                                  w1a, w1b, w3a, w3b, w2a, w2b)
        lo_outs.append(_return_gather(out_lo_c, dest2_flat[c, 0:1]))
        hi_outs.append(_return_gather(out_hi_c, dest2_flat[c, 0:1]))
    out = jnp.concatenate([jnp.stack(lo_outs), jnp.stack(hi_outs)], axis=-1)
    return out.reshape(bsz, seq_len, dim)


# speedup vs baseline: 1.3098x; 1.2151x over previous
import functools

import jax
import jax.numpy as jnp
from jax.experimental import pallas as pl
from jax.experimental.pallas import tpu as pltpu
from jax.experimental.pallas import tpu_sc as plsc

_DIM = 768
_HALF = _DIM // 2
_HIDDEN = 2048
_BLK = 512
_CHUNK = 8192
_NBLK = _CHUNK // _BLK + 1
_PAD_CHUNK = _NBLK * _BLK
_W = 128


@functools.cache
def _vector_mesh():
    return plsc.VectorSubcoreMesh(
        core_axis_name="core", subcore_axis_name="subcore")


def _routing_kernel(ids_ref, dest2_ref, eblk_ref):
    ind1 = ids_ref[0]
    ind0 = 1.0 - ind1
    r128 = jax.lax.broadcasted_iota(jnp.int32, (128, 128), 0)
    c128 = jax.lax.broadcasted_iota(jnp.int32, (128, 128), 1)
    upper = (r128 <= c128).astype(jnp.bfloat16)
    r64 = jax.lax.broadcasted_iota(jnp.int32, (64, 64), 0)
    c64 = jax.lax.broadcasted_iota(jnp.int32, (64, 64), 1)
    strict_lower = (r64 > c64).astype(jnp.bfloat16)

    def excl_prefix(ind):
        cs = jax.lax.dot_general(ind.astype(jnp.bfloat16), upper,
                                 (((1,), (0,)), ((), ())),
                                 preferred_element_type=jnp.float32)
        tot = cs[:, 127:128]
        rp = jax.lax.dot_general(strict_lower, tot.astype(jnp.bfloat16),
                                 (((1,), (0,)), ((), ())),
                                 preferred_element_type=jnp.float32)
        return rp + cs - ind, tot, rp

    g0, tot0, rp0 = excl_prefix(ind0)
    g1, _, _ = excl_prefix(ind1)
    n0 = (rp0[63:64, 0:1] + tot0[63:64, 0:1]).astype(jnp.int32)
    nb0 = (n0 + (_BLK - 1)) >> 9
    seg1_start = nb0 << 9
    dest = jnp.where(ind1 > 0.5,
                     seg1_start + g1.astype(jnp.int32),
                     g0.astype(jnp.int32))
    dest2_ref[0, 0] = dest
    dest2_ref[0, 1] = dest + _PAD_CHUNK
    lane = jax.lax.broadcasted_iota(jnp.int32, (1, 128), 1)
    eblk_ref[0] = (lane >= nb0).astype(jnp.int32)


def _route(ids_f32):
    n_chunks = ids_f32.shape[0]
    return pl.pallas_call(
        _routing_kernel,
        grid=(n_chunks,),
        in_specs=[pl.BlockSpec((1, 64, 128), lambda c: (c, 0, 0))],
        out_specs=[pl.BlockSpec((1, 2, 64, 128), lambda c: (c, 0, 0, 0)),
                   pl.BlockSpec((1, 1, 128), lambda c: (c, 0, 0))],
        out_shape=[jax.ShapeDtypeStruct((n_chunks, 2, 64, 128), jnp.int32),
                   jax.ShapeDtypeStruct((n_chunks, 1, 128), jnp.int32)],
    )(ids_f32)


def _dispatch(xf, dest2_c, chunk_idx):
    blk_off = chunk_idx * (_CHUNK // _W)

    @pl.kernel(out_type=jax.ShapeDtypeStruct((2 * _PAD_CHUNK, _HALF),
                                             jnp.float32),
               mesh=_vector_mesh())
    def dispatch_kernel(x_hbm, d_hbm, o_hbm):
        def body(x_vmem, i_vmem):
            pltpu.sync_copy(x_vmem, o_hbm.at[i_vmem.at[0]])

        pltpu.emit_pipeline(
            body,
            grid=(2, _CHUNK // _W),
            in_specs=[pl.BlockSpec((_W, _HALF),
                                   lambda p, i: (blk_off + i, p)),
                      pl.BlockSpec((1, _W), lambda p, i: (p, i))],
            out_specs=[],
            core_axis_name=("core", "subcore"),
            dimension_semantics=(pltpu.PARALLEL, pltpu.PARALLEL),
        )(x_hbm, d_hbm)

    return dispatch_kernel(xf, dest2_c)


def _return_gather(out_plane_c, dest_c):
    @pl.kernel(out_type=jax.ShapeDtypeStruct((_CHUNK, _HALF), jnp.float32),
               mesh=_vector_mesh())
    def return_kernel(src_hbm, d_hbm, o_hbm):
        def body(i_vmem, o_vmem):
            pltpu.sync_copy(src_hbm.at[i_vmem.at[0]], o_vmem)

        pltpu.emit_pipeline(
            body,
            grid=(_CHUNK // _W,),
            in_specs=[pl.BlockSpec((1, _W), lambda i: (0, i))],
            out_specs=[pl.BlockSpec((_W, _HALF), lambda i: (i, 0))],
            core_axis_name=("core", "subcore"),
            dimension_semantics=(pltpu.PARALLEL,),
        )(d_hbm, o_hbm)

    return return_kernel(out_plane_c, dest_c)


def _ffn_kernel(eblk_ref, xlo_ref, xhi_ref, w1_ref, w3_ref, w2_ref,
                olo_ref, ohi_ref):
    xb = jnp.concatenate([xlo_ref[...], xhi_ref[...]],
                         axis=1).astype(jnp.bfloat16)
    dims = (((1,), (1,)), ((), ()))
    x1 = jax.lax.dot_general(xb, w1_ref[0], dims,
                             preferred_element_type=jnp.float32)
    x3 = jax.lax.dot_general(xb, w3_ref[0], dims,
                             preferred_element_type=jnp.float32)
    h = (x1 * jax.lax.logistic(x1) * x3).astype(jnp.bfloat16)
    out = jax.lax.dot_general(h, w2_ref[0], dims,
                              preferred_element_type=jnp.float32)
    olo_ref[...] = out[:, :_HALF]
    ohi_ref[...] = out[:, _HALF:]


def _ffn(x_sorted_c, eblk_c, w1, w3, w2):
    grid_spec = pltpu.PrefetchScalarGridSpec(
        num_scalar_prefetch=1,
        grid=(_NBLK,),
        in_specs=[
            pl.BlockSpec((_BLK, _HALF), lambda j, s: (j, 0)),
            pl.BlockSpec((_BLK, _HALF), lambda j, s: (_NBLK + j, 0)),
            pl.BlockSpec((1, _HIDDEN, _DIM), lambda j, s: (s[j], 0, 0)),
            pl.BlockSpec((1, _HIDDEN, _DIM), lambda j, s: (s[j], 0, 0)),
            pl.BlockSpec((1, _DIM, _HIDDEN), lambda j, s: (s[j], 0, 0)),
        ],
        out_specs=[pl.BlockSpec((_BLK, _HALF), lambda j, s: (j, 0)),
                   pl.BlockSpec((_BLK, _HALF), lambda j, s: (j, 0))],
    )
    return pl.pallas_call(
        _ffn_kernel,
        grid_spec=grid_spec,
        out_shape=[jax.ShapeDtypeStruct((_PAD_CHUNK, _HALF), jnp.float32),
                   jax.ShapeDtypeStruct((_PAD_CHUNK, _HALF), jnp.float32)],
    )(eblk_c, x_sorted_c, x_sorted_c, w1, w3, w2)


def kernel(x, modality_ids, W1, W2, W3):
    bsz, seq_len, dim = x.shape
    n_tok = bsz * seq_len
    n_chunks = n_tok // _CHUNK
    xf = x.reshape(n_tok, dim)
    ids_f32 = modality_ids.astype(jnp.float32).reshape(n_chunks, 64, 128)
    w1 = W1.astype(jnp.bfloat16)
    w3 = W3.astype(jnp.bfloat16)
    w2 = W2.astype(jnp.bfloat16)

    dest2, eblk = _route(ids_f32)
    dest2_flat = dest2.reshape(n_chunks, 2, _CHUNK)
    eblk_flat = eblk.reshape(n_chunks, 128)

    lo_outs, hi_outs = [], []
    for c in range(n_chunks):
        x_sorted_c = _dispatch(xf, dest2_flat[c], c)
        out_lo_c, out_hi_c = _ffn(x_sorted_c, eblk_flat[c], w1, w3, w2)
        lo_outs.append(_return_gather(out_lo_c, dest2_flat[c, 0:1]))
        hi_outs.append(_return_gather(out_hi_c, dest2_flat[c, 0:1]))
    out = jnp.concatenate([jnp.stack(lo_outs), jnp.stack(hi_outs)], axis=-1)
    return out.reshape(bsz, seq_len, dim)


# speedup vs baseline: 1.3190x; 1.0070x over previous
import functools

import jax
import jax.numpy as jnp
from jax.experimental import pallas as pl
from jax.experimental.pallas import tpu as pltpu
from jax.experimental.pallas import tpu_sc as plsc

_DIM = 768
_HALF = _DIM // 2
_HIDDEN = 2048
_BLK = 1024
_CHUNK = 8192
_NBLK = _CHUNK // _BLK + 1
_PAD_CHUNK = _NBLK * _BLK
_W = 128
_BLK_LOG2 = _BLK.bit_length() - 1
assert (1 << _BLK_LOG2) == _BLK


@functools.cache
def _vector_mesh():
    return plsc.VectorSubcoreMesh(
        core_axis_name="core", subcore_axis_name="subcore")


def _routing_kernel(ids_ref, dest2_ref, eblk_ref):
    ind1 = ids_ref[0]
    ind0 = 1.0 - ind1
    r128 = jax.lax.broadcasted_iota(jnp.int32, (128, 128), 0)
    c128 = jax.lax.broadcasted_iota(jnp.int32, (128, 128), 1)
    upper = (r128 <= c128).astype(jnp.bfloat16)
    r64 = jax.lax.broadcasted_iota(jnp.int32, (64, 64), 0)
    c64 = jax.lax.broadcasted_iota(jnp.int32, (64, 64), 1)
    strict_lower = (r64 > c64).astype(jnp.bfloat16)

    def excl_prefix(ind):
        cs = jax.lax.dot_general(ind.astype(jnp.bfloat16), upper,
                                 (((1,), (0,)), ((), ())),
                                 preferred_element_type=jnp.float32)
        tot = cs[:, 127:128]
        rp = jax.lax.dot_general(strict_lower, tot.astype(jnp.bfloat16),
                                 (((1,), (0,)), ((), ())),
                                 preferred_element_type=jnp.float32)
        return rp + cs - ind, tot, rp

    g0, tot0, rp0 = excl_prefix(ind0)
    g1, _, _ = excl_prefix(ind1)
    n0 = (rp0[63:64, 0:1] + tot0[63:64, 0:1]).astype(jnp.int32)
    nb0 = (n0 + (_BLK - 1)) >> _BLK_LOG2
    seg1_start = nb0 << _BLK_LOG2
    dest = jnp.where(ind1 > 0.5,
                     seg1_start + g1.astype(jnp.int32),
                     g0.astype(jnp.int32))
    dest2_ref[0, 0] = dest
    dest2_ref[0, 1] = dest + _PAD_CHUNK
    lane = jax.lax.broadcasted_iota(jnp.int32, (1, 128), 1)
    eblk_ref[0] = (lane >= nb0).astype(jnp.int32)


def _route(ids_f32):
    n_chunks = ids_f32.shape[0]
    return pl.pallas_call(
        _routing_kernel,
        grid=(n_chunks,),
        in_specs=[pl.BlockSpec((1, 64, 128), lambda c: (c, 0, 0))],
        out_specs=[pl.BlockSpec((1, 2, 64, 128), lambda c: (c, 0, 0, 0)),
                   pl.BlockSpec((1, 1, 128), lambda c: (c, 0, 0))],
        out_shape=[jax.ShapeDtypeStruct((n_chunks, 2, 64, 128), jnp.int32),
                   jax.ShapeDtypeStruct((n_chunks, 1, 128), jnp.int32)],
    )(ids_f32)


def _dispatch(xf, dest2_c, chunk_idx):
    blk_off = chunk_idx * (_CHUNK // _W)

    @pl.kernel(out_type=jax.ShapeDtypeStruct((2 * _PAD_CHUNK, _HALF),
                                             jnp.float32),
               mesh=_vector_mesh())
    def dispatch_kernel(x_hbm, d_hbm, o_hbm):
        def body(x_vmem, i_vmem):
            pltpu.sync_copy(x_vmem, o_hbm.at[i_vmem.at[0]])

        pltpu.emit_pipeline(
            body,
            grid=(2, _CHUNK // _W),
            in_specs=[pl.BlockSpec((_W, _HALF),
                                   lambda p, i: (blk_off + i, p)),
                      pl.BlockSpec((1, _W), lambda p, i: (p, i))],
            out_specs=[],
            core_axis_name=("core", "subcore"),
            dimension_semantics=(pltpu.PARALLEL, pltpu.PARALLEL),
        )(x_hbm, d_hbm)

    return dispatch_kernel(xf, dest2_c)


def _return_gather(out_plane_c, dest_c):
    @pl.kernel(out_type=jax.ShapeDtypeStruct((_CHUNK, _HALF), jnp.float32),
               mesh=_vector_mesh())
    def return_kernel(src_hbm, d_hbm, o_hbm):
        def body(i_vmem, o_vmem):
            pltpu.sync_copy(src_hbm.at[i_vmem.at[0]], o_vmem)

        pltpu.emit_pipeline(
            body,
            grid=(_CHUNK // _W,),
            in_specs=[pl.BlockSpec((1, _W), lambda i: (0, i))],
            out_specs=[pl.BlockSpec((_W, _HALF), lambda i: (i, 0))],
            core_axis_name=("core", "subcore"),
            dimension_semantics=(pltpu.PARALLEL,),
        )(d_hbm, o_hbm)

    return return_kernel(out_plane_c, dest_c)


def _wcast_kernel(w1_ref, w3_ref, w2_ref, o1_ref, o3_ref, o2_ref):
    o1_ref[...] = w1_ref[...].astype(jnp.bfloat16)
    o3_ref[...] = w3_ref[...].astype(jnp.bfloat16)
    o2_ref[...] = w2_ref[...].astype(jnp.bfloat16)


def _wcast(W1, W3, W2):
    kq = _HIDDEN // 4
    return pl.pallas_call(
        _wcast_kernel,
        grid=(2, 4),
        in_specs=[
            pl.BlockSpec((1, kq, _DIM), lambda e, k: (e, k, 0)),
            pl.BlockSpec((1, kq, _DIM), lambda e, k: (e, k, 0)),
            pl.BlockSpec((1, _DIM, kq), lambda e, k: (e, 0, k)),
        ],
        out_specs=[
            pl.BlockSpec((1, kq, _DIM), lambda e, k: (e, k, 0)),
            pl.BlockSpec((1, kq, _DIM), lambda e, k: (e, k, 0)),
            pl.BlockSpec((1, _DIM, kq), lambda e, k: (e, 0, k)),
        ],
        out_shape=[
            jax.ShapeDtypeStruct((2, _HIDDEN, _DIM), jnp.bfloat16),
            jax.ShapeDtypeStruct((2, _HIDDEN, _DIM), jnp.bfloat16),
            jax.ShapeDtypeStruct((2, _DIM, _HIDDEN), jnp.bfloat16),
        ],
    )(W1, W3, W2)


def _ffn_kernel(eblk_ref, xlo_ref, xhi_ref, w1_ref, w3_ref, w2_ref,
                olo_ref, ohi_ref):
    xb = jnp.concatenate([xlo_ref[...], xhi_ref[...]],
                         axis=1).astype(jnp.bfloat16)
    dims = (((1,), (1,)), ((), ()))
    x1 = jax.lax.dot_general(xb, w1_ref[0], dims,
                             preferred_element_type=jnp.float32)
    x3 = jax.lax.dot_general(xb, w3_ref[0], dims,
                             preferred_element_type=jnp.float32)
    h = (x1 * jax.lax.logistic(x1) * x3).astype(jnp.bfloat16)
    out = jax.lax.dot_general(h, w2_ref[0], dims,
                              preferred_element_type=jnp.float32)
    olo_ref[...] = out[:, :_HALF]
    ohi_ref[...] = out[:, _HALF:]


def _ffn(x_sorted_c, eblk_c, w1, w3, w2):
    grid_spec = pltpu.PrefetchScalarGridSpec(
        num_scalar_prefetch=1,
        grid=(_NBLK,),
        in_specs=[
            pl.BlockSpec((_BLK, _HALF), lambda j, s: (j, 0)),
            pl.BlockSpec((_BLK, _HALF), lambda j, s: (_NBLK + j, 0)),
            pl.BlockSpec((1, _HIDDEN, _DIM), lambda j, s: (s[j], 0, 0)),
            pl.BlockSpec((1, _HIDDEN, _DIM), lambda j, s: (s[j], 0, 0)),
            pl.BlockSpec((1, _DIM, _HIDDEN), lambda j, s: (s[j], 0, 0)),
        ],
        out_specs=[pl.BlockSpec((_BLK, _HALF), lambda j, s: (j, 0)),
                   pl.BlockSpec((_BLK, _HALF), lambda j, s: (j, 0))],
    )
    return pl.pallas_call(
        _ffn_kernel,
        grid_spec=grid_spec,
        out_shape=[jax.ShapeDtypeStruct((_PAD_CHUNK, _HALF), jnp.float32),
                   jax.ShapeDtypeStruct((_PAD_CHUNK, _HALF), jnp.float32)],
    )(eblk_c, x_sorted_c, x_sorted_c, w1, w3, w2)


def kernel(x, modality_ids, W1, W2, W3):
    bsz, seq_len, dim = x.shape
    n_tok = bsz * seq_len
    n_chunks = n_tok // _CHUNK
    xf = x.reshape(n_tok, dim)
    ids_f32 = modality_ids.astype(jnp.float32).reshape(n_chunks, 64, 128)
    w1, w3, w2 = _wcast(W1, W3, W2)

    dest2, eblk = _route(ids_f32)
    dest2_flat = dest2.reshape(n_chunks, 2, _CHUNK)
    eblk_flat = eblk.reshape(n_chunks, 128)

    lo_outs, hi_outs = [], []
    for c in range(n_chunks):
        x_sorted_c = _dispatch(xf, dest2_flat[c], c)
        out_lo_c, out_hi_c = _ffn(x_sorted_c, eblk_flat[c], w1, w3, w2)
        lo_outs.append(_return_gather(out_lo_c, dest2_flat[c, 0:1]))
        hi_outs.append(_return_gather(out_hi_c, dest2_flat[c, 0:1]))
    out = jnp.concatenate([jnp.stack(lo_outs), jnp.stack(hi_outs)], axis=-1)
    return out.reshape(bsz, seq_len, dim)
